# Initial kernel scaffold; baseline (speedup 1.0000x reference)
import functools

import jax
import jax.numpy as jnp
from jax import lax
from jax.experimental import pallas as pl
from jax.experimental.pallas import tpu as pltpu
from jax.experimental.pallas import tpu_sc as plsc

_IN_CHANNELS = 26
_OUT_CHANNELS = 64
_NUM_STEPS = 3
_NUM_POINTS = 16
_BATCH = 16384

_NUM_CORES = 2
_NUM_SUBCORES = 16
_NUM_WORKERS = _NUM_CORES * _NUM_SUBCORES
_ROWS_PER = _BATCH // _NUM_WORKERS
_LANES = 16
_GROUPS = _ROWS_PER // _LANES


def _tables_body(angles_ref, velo_ref, vc_ref, vs_ref):
    a = angles_ref[...]
    v = velo_ref[...]
    vc_ref[...] = v * jnp.cos(a)
    vs_ref[...] = v * jnp.sin(a)


_make_tables = pl.pallas_call(
    _tables_body,
    out_shape=(
        jax.ShapeDtypeStruct((_NUM_POINTS,), jnp.float32),
        jax.ShapeDtypeStruct((_NUM_POINTS,), jnp.float32),
    ),
)


@functools.partial(
    pl.kernel,
    mesh=plsc.VectorSubcoreMesh(core_axis_name="c", subcore_axis_name="s"),
    out_type=jax.ShapeDtypeStruct((_BATCH, _OUT_CHANNELS), jnp.float32),
    scratch_types=[
        pltpu.VMEM((_ROWS_PER, _IN_CHANNELS), jnp.float32),
        pltpu.VMEM((_ROWS_PER, _OUT_CHANNELS), jnp.float32),
        pltpu.VMEM((_NUM_POINTS,), jnp.float32),
        pltpu.VMEM((_NUM_POINTS,), jnp.float32),
    ],
)
def _sc_flow(data_hbm, vc_hbm, vs_hbm, out_hbm, data_v, out_v, vc_v, vs_v):
    wid = lax.axis_index("s") * _NUM_CORES + lax.axis_index("c")
    base = wid * _ROWS_PER
    pltpu.sync_copy(data_hbm.at[pl.ds(base, _ROWS_PER), :], data_v)
    pltpu.sync_copy(vc_hbm, vc_v)
    pltpu.sync_copy(vs_hbm, vs_v)
    lane = lax.iota(jnp.int32, _LANES)

    def g_body(g, carry):
        rows = g * _LANES + lane
        acc = jnp.zeros((_LANES,), jnp.float32)
        for c in range(_IN_CHANNELS):
            cvec = jnp.full((_LANES,), c, jnp.int32)
            x = plsc.load_gather(data_v, [rows, cvec])
            for _ in range(_NUM_STEPS):
                e = jnp.exp(x + x)
                t = 1.0 - 2.0 / (e + 1.0)
                pf = (1.0 + t) * (_NUM_POINTS / 2.0)
                p = (pf + 0.5).astype(jnp.int32)
                p = jnp.minimum(jnp.maximum(p, 0), _NUM_POINTS - 1)
                v = plsc.load_gather(vc_v, [p])
                w = plsc.load_gather(vs_v, [p])
                x = x + (v + t * w) * (1.0 / _NUM_STEPS)
            acc = acc + x
        for col in range(_OUT_CHANNELS):
            colvec = jnp.full((_LANES,), col, jnp.int32)
            plsc.store_scatter(out_v, [rows, colvec], acc)
        return carry

    lax.fori_loop(0, _GROUPS, g_body, 0)
    pltpu.sync_copy(out_v, out_hbm.at[pl.ds(base, _ROWS_PER), :])


def kernel(data, angles, velo):
    vc, vs = _make_tables(angles, velo)
    return _sc_flow(data, vc, vs)


# speedup vs baseline: 1.8128x; 1.8128x over previous
import functools

import jax
import jax.numpy as jnp
from jax import lax
from jax.experimental import pallas as pl
from jax.experimental.pallas import tpu as pltpu
from jax.experimental.pallas import tpu_sc as plsc

_IN_CHANNELS = 26
_OUT_CHANNELS = 64
_NUM_STEPS = 3
_NUM_POINTS = 16
_BATCH = 16384

_NUM_CORES = 2
_NUM_SUBCORES = 16
_NUM_WORKERS = _NUM_CORES * _NUM_SUBCORES
_ROWS_PER = _BATCH // _NUM_WORKERS
_LANES = 16
_GROUPS = _ROWS_PER // _LANES


def _tables_body(angles_ref, velo_ref, vc_ref, vs_ref):
    a = angles_ref[...]
    v = velo_ref[...]
    vc_ref[...] = v * jnp.cos(a)
    vs_ref[...] = v * jnp.sin(a)


_make_tables = pl.pallas_call(
    _tables_body,
    out_shape=(
        jax.ShapeDtypeStruct((_NUM_POINTS,), jnp.float32),
        jax.ShapeDtypeStruct((_NUM_POINTS,), jnp.float32),
    ),
)


_IN_PER = _ROWS_PER * _IN_CHANNELS
_OUT_PER = _ROWS_PER * _OUT_CHANNELS


@functools.partial(
    pl.kernel,
    mesh=plsc.VectorSubcoreMesh(core_axis_name="c", subcore_axis_name="s"),
    out_type=jax.ShapeDtypeStruct((_BATCH * _OUT_CHANNELS,), jnp.float32),
    compiler_params=pltpu.CompilerParams(needs_layout_passes=False),
    scratch_types=[
        pltpu.VMEM((_IN_PER,), jnp.float32),
        pltpu.VMEM((_OUT_PER,), jnp.float32),
        pltpu.VMEM((_NUM_POINTS,), jnp.float32),
        pltpu.VMEM((_NUM_POINTS,), jnp.float32),
    ],
)
def _sc_flow(data_hbm, vc_hbm, vs_hbm, out_hbm, data_v, out_v, vc_v, vs_v):
    wid = lax.axis_index("s") * _NUM_CORES + lax.axis_index("c")
    pltpu.sync_copy(data_hbm.at[pl.ds(wid * _IN_PER, _IN_PER)], data_v)
    pltpu.sync_copy(vc_hbm, vc_v)
    pltpu.sync_copy(vs_hbm, vs_v)
    lane = lax.iota(jnp.int32, _LANES)

    def g_body(g, carry):
        row0 = g * _LANES
        in_idx = (row0 * _IN_CHANNELS) + lane * _IN_CHANNELS
        out_idx = (row0 * _OUT_CHANNELS) + lane * _OUT_CHANNELS
        acc = jnp.zeros((_LANES,), jnp.float32)
        for c in range(_IN_CHANNELS):
            x = plsc.load_gather(data_v, [in_idx + c])
            for _ in range(_NUM_STEPS):
                e = jnp.exp(x + x)
                t = 1.0 - 2.0 / (e + 1.0)
                pf = (1.0 + t) * (_NUM_POINTS / 2.0)
                p = (pf + 0.5).astype(jnp.int32)
                p = jnp.minimum(jnp.maximum(p, 0), _NUM_POINTS - 1)
                v = plsc.load_gather(vc_v, [p])
                w = plsc.load_gather(vs_v, [p])
                x = x + (v + t * w) * (1.0 / _NUM_STEPS)
            acc = acc + x
        for col in range(_OUT_CHANNELS):
            plsc.store_scatter(out_v, [out_idx + col], acc)
        return carry

    lax.fori_loop(0, _GROUPS, g_body, 0)
    pltpu.sync_copy(out_v, out_hbm.at[pl.ds(wid * _OUT_PER, _OUT_PER)])


def kernel(data, angles, velo):
    vc, vs = _make_tables(angles, velo)
    flat = _sc_flow(data.reshape(-1), vc, vs)
    return flat.reshape(_BATCH, _OUT_CHANNELS)


# speedup vs baseline: 3.3725x; 1.8604x over previous
import functools

import jax
import jax.numpy as jnp
from jax import lax
from jax.experimental import pallas as pl
from jax.experimental.pallas import tpu as pltpu
from jax.experimental.pallas import tpu_sc as plsc

_IN_CHANNELS = 26
_OUT_CHANNELS = 64
_NUM_STEPS = 3
_NUM_POINTS = 16
_BATCH = 16384

_NUM_CORES = 2
_NUM_SUBCORES = 16
_NUM_WORKERS = _NUM_CORES * _NUM_SUBCORES
_ROWS_PER = _BATCH // _NUM_WORKERS
_LANES = 16
_GROUPS = _ROWS_PER // _LANES


_TAB = 32


def _tables_body(angles_ref, velo_ref, vc_ref, vs_ref):
    a = angles_ref[...]
    v = velo_ref[...]
    scale = 1.0 / _NUM_STEPS
    vc_ref[...] = v * jnp.cos(a) * scale
    vs_ref[...] = v * jnp.sin(a) * scale


_make_tables = pl.pallas_call(
    _tables_body,
    out_shape=(
        jax.ShapeDtypeStruct((_TAB,), jnp.float32),
        jax.ShapeDtypeStruct((_TAB,), jnp.float32),
    ),
)


_IN_PER = _ROWS_PER * _IN_CHANNELS
_OUT_PER = _ROWS_PER * _OUT_CHANNELS


@functools.partial(
    pl.kernel,
    mesh=plsc.VectorSubcoreMesh(core_axis_name="c", subcore_axis_name="s"),
    out_type=jax.ShapeDtypeStruct((_BATCH * _OUT_CHANNELS,), jnp.float32),
    compiler_params=pltpu.CompilerParams(needs_layout_passes=False),
    scratch_types=[
        pltpu.VMEM((_IN_PER,), jnp.float32),
        pltpu.VMEM((_OUT_PER,), jnp.float32),
        pltpu.VMEM((_TAB,), jnp.float32),
        pltpu.VMEM((_TAB,), jnp.float32),
    ],
)
def _sc_flow(data_hbm, vc_hbm, vs_hbm, out_hbm, data_v, out_v, vc_v, vs_v):
    wid = lax.axis_index("s") * _NUM_CORES + lax.axis_index("c")
    pltpu.sync_copy(data_hbm.at[pl.ds(wid * _IN_PER, _IN_PER)], data_v)
    pltpu.sync_copy(vc_hbm, vc_v)
    pltpu.sync_copy(vs_hbm, vs_v)
    lane = lax.iota(jnp.int32, _LANES)
    lane_in = lane * _IN_CHANNELS
    lane_out = lane * _OUT_CHANNELS

    _CHUNK = 13

    def g_body(g, carry):
        row0 = g * _LANES
        in_idx = (row0 * _IN_CHANNELS) + lane_in
        out_idx = (row0 * _OUT_CHANNELS) + lane_out
        acc = jnp.zeros((_LANES,), jnp.float32)
        for lo in range(0, _IN_CHANNELS, _CHUNK):
            cs = range(lo, min(lo + _CHUNK, _IN_CHANNELS))
            xs = [plsc.load_gather(data_v, [in_idx + c]) for c in cs]
            for _ in range(_NUM_STEPS):
                es = [jnp.exp(x + x) for x in xs]
                rs = [2.0 / (e + 1.0) for e in es]
                ts = [1.0 - r for r in rs]
                pfs = [16.5 - 8.0 * r for r in rs]
                ps = [pf.astype(jnp.int32) for pf in pfs]
                vs_ = [plsc.load_gather(vc_v, [p]) for p in ps]
                ws = [plsc.load_gather(vs_v, [p]) for p in ps]
                xs = [x + (v + t * w)
                      for x, t, v, w in zip(xs, ts, vs_, ws)]
            for x in xs:
                acc = acc + x
        for col in range(_OUT_CHANNELS):
            plsc.store_scatter(out_v, [out_idx + col], acc)
        return carry

    lax.fori_loop(0, _GROUPS, g_body, 0)
    pltpu.sync_copy(out_v, out_hbm.at[pl.ds(wid * _OUT_PER, _OUT_PER)])


def kernel(data, angles, velo):
    angles_p = jnp.concatenate([angles, jnp.broadcast_to(angles[-1:], (_TAB - _NUM_POINTS,))])
    velo_p = jnp.concatenate([velo, jnp.broadcast_to(velo[-1:], (_TAB - _NUM_POINTS,))])
    vc, vs = _make_tables(angles_p, velo_p)
    flat = _sc_flow(data.reshape(-1), vc, vs)
    return flat.reshape(_BATCH, _OUT_CHANNELS)
